# baseline (speedup 1.0000x reference)
import functools

import jax
import jax.numpy as jnp
from jax import lax
from jax.experimental import pallas as pl
from jax.experimental.pallas import tpu as pltpu
from jax.experimental.pallas import tpu_sc as plsc

N = 10000
D = 128
R = 200
E = 320000

L = 16
NC = 2
NS = 16
NW = NC * NS
NP = 10240
EP = 327680
ET = EP // NW
CB = 128
NCH = EP // CB
CHT = NCH // NS
NPR = NP // L
DH = D // 2


def _dense_x_body(x_ref, w_ref, aw_ref, xa_ref, s1_ref, s2_ref):
    xb = x_ref[...]
    w1 = w_ref[pl.ds(0, D), :]
    xa_ref[...] = jnp.dot(xb, w1, preferred_element_type=jnp.float32)
    a1 = aw_ref[pl.ds(0, D), :]
    a2 = aw_ref[pl.ds(D, D), :]
    s1_ref[...] = jnp.dot(xb, a1, preferred_element_type=jnp.float32)
    s2_ref[...] = jnp.dot(xb, a2, preferred_element_type=jnp.float32)


def _dense_rel_body(e_ref, w_ref, aw_ref, ra_ref, sr_ref):
    eb = e_ref[...]
    w2 = w_ref[pl.ds(D, D), :]
    ra_ref[...] = jnp.dot(eb, w2, preferred_element_type=jnp.float32)
    a3 = aw_ref[pl.ds(2 * D, D), :]
    sr_ref[...] = jnp.dot(eb, a3, preferred_element_type=jnp.float32)


def _pass1_body(srcf, dstf, typf, timf, s1h, s2h, srh, dlh, idxh,
                ex_out, den_out,
                sv, dv, tv, tm, s1r, s2r, srr, dlr, exr, denr, markr,
                idxr, zbuf, dsh):
    c = lax.axis_index("c")
    s = lax.axis_index("s")
    wid = c * NS + s
    base = wid * ET

    pltpu.sync_copy(srcf.at[pl.ds(base, ET)], sv)
    pltpu.sync_copy(dstf.at[pl.ds(base, ET)], dv)
    pltpu.sync_copy(typf.at[pl.ds(base, ET)], tv)
    pltpu.sync_copy(timf.at[pl.ds(base, ET)], tm)
    pltpu.sync_copy(s1h, s1r)
    pltpu.sync_copy(s2h, s2r)
    pltpu.sync_copy(srh, srr)
    pltpu.sync_copy(dlh, dlr)
    pltpu.sync_copy(idxh, idxr)

    zz = jnp.zeros((L,), jnp.float32)

    @plsc.parallel_loop(0, NPR)
    def zbody(i):
        denr[i, pl.ds(0, L)] = zz

    for q in range(40):
        zbuf[q, pl.ds(0, L)] = zz

    iot = lax.iota(jnp.int32, L)
    dlv = dlr[pl.ds(0, L)]

    def body(i, carry):
        sl = pl.ds(i * L, L)
        svv = sv[sl]
        dvv = dv[sl]
        tvv = tv[sl]
        tmv = tm[sl]
        a = plsc.load_gather(s1r, [svv])
        b = plsc.load_gather(s2r, [dvv])
        cc = plsc.load_gather(srr, [tvv])
        z = a + b + cc
        e = jnp.where(z >= 0.0, z, z * jnp.float32(0.01))
        ex = jnp.exp(-(tmv * dlv) * e)
        exr[sl] = ex
        dhi = lax.shift_right_logical(dvv, 4)
        dlo = lax.bitwise_and(dvv, 15)

        def one(active):
            am = active != 0
            plsc.store_scatter(markr, [dvv], iot, mask=am)
            got = plsc.load_gather(markr, [dvv], mask=am)
            win = jnp.logical_and(am, got == iot)
            plsc.addupdate_scatter(denr, [dhi, dlo], ex, mask=win)
            return jnp.where(win, 0, active)

        active = one(jnp.ones((L,), jnp.int32))
        active = lax.while_loop(lambda a_: jnp.max(a_) > 0, one, active)
        return carry

    lax.fori_loop(0, ET // L, body, 0)

    pltpu.sync_copy(exr, ex_out.at[pl.ds(base, ET)])

    pltpu.sync_copy(zbuf, dsh.at[pl.ds(s * 40, 40)])
    plsc.subcore_barrier()
    for q in range(5):
        pltpu.sync_copy(denr.at[pl.ds(q * 128, 128)], dsh.at[idxr.at[q]],
                        add=True)
    plsc.subcore_barrier()
    pltpu.sync_copy(dsh.at[pl.ds(s * 40, 40)], den_out.at[c, pl.ds(s * 40, 40)])


def _bcast_lane(v, l):
    idx = jnp.full((L, 1), l, jnp.int32)
    dn = lax.GatherDimensionNumbers(offset_dims=(), collapsed_slice_dims=(0,),
                                    start_index_map=(0,))
    return lax.gather(v, idx, dn, (1,),
                      mode=lax.GatherScatterMode.PROMISE_IN_BOUNDS)


def _pass2_body(metah, denh, ra2f, xah,
                hpart,
                m0, m1, m2, m3, d0, d1, d2, d3, winvr, dtmp, rar,
                g0, g1, g2, g3,
                ms0, ms1, ms2, ms3, gs0, gs1, gs2, gs3,
                ss0, ss1, ss2, ss3, hsh):
    c = lax.axis_index("c")
    s = lax.axis_index("s")
    mb = [m0, m1, m2, m3]
    db = [d0, d1, d2, d3]
    gb = [g0, g1, g2, g3]
    msem = [ms0, ms1, ms2, ms3]
    gsem = [gs0, gs1, gs2, gs3]
    ssem = [ss0, ss1, ss2, ss3]

    zz = jnp.zeros((L,), jnp.float32)

    def zb(r, carry):
        for k in range(DH // L):
            g0[r, pl.ds(k * L, L)] = zz
        return carry

    lax.fori_loop(0, CB, zb, 0)
    for q in range(5):
        pltpu.sync_copy(g0, hsh.at[pl.ds(s * 640 + q * CB, CB)])
    plsc.subcore_barrier()

    pltpu.sync_copy(denh.at[0], winvr)
    pltpu.sync_copy(denh.at[1], dtmp)
    pltpu.sync_copy(ra2f.at[pl.ds(c * (R * DH), R * DH)], rar)
    one16 = jnp.ones((L,), jnp.float32)

    @plsc.parallel_loop(0, NPR)
    def winv_body(i):
        d = winvr[i, pl.ds(0, L)] + dtmp[i, pl.ds(0, L)]
        winvr[i, pl.ds(0, L)] = one16 / jnp.where(d > 0.0, d, one16)

    iot = lax.iota(jnp.int32, L)

    def meta_copy(j, mbuf, msem):
        return pltpu.make_async_copy(
            metah.at[pl.ds((s * CHT + j) * 4, 4)], mbuf, msem)

    def gather_copy(mbuf, gbuf, gsem):
        return pltpu.make_async_copy(xah.at[c].at[mbuf.at[0]], gbuf, gsem)

    def scat_copy(dbuf, gbuf, ssem):
        return pltpu.make_async_copy(gbuf, hsh.at[dbuf.at[0]], ssem)

    def compute(mbuf, dbuf, gbuf):
        @plsc.parallel_loop(0, CB // L)
        def grp(g):
            gl = pl.ds(g * L, L)
            dvv = mbuf[1, gl]
            dbuf[0, gl] = dvv
            tvv = mbuf[2, gl]
            exv = plsc.bitcast(mbuf[3, gl], jnp.float32)
            dhi = lax.shift_right_logical(dvv, 4)
            dlo = lax.bitwise_and(dvv, 15)
            wv = exv * plsc.load_gather(winvr, [dhi, dlo])
            for l in range(L):
                wb = _bcast_lane(wv, l)
                tb = _bcast_lane(tvv, l)
                r = g * L + l
                for k in range(DH // L):
                    idx = tb * DH + (k * L + iot)
                    rc = plsc.load_gather(rar, [idx])
                    gv = gbuf[r, pl.ds(k * L, L)]
                    gbuf[r, pl.ds(k * L, L)] = wb * (gv + rc)

    for r in range(4):
        meta_copy(r, mb[r], msem[r]).start()
    for r in range(2):
        meta_copy(r, mb[r], msem[r]).wait()
        gather_copy(mb[r], gb[r], gsem[r]).start()

    def quad(q, carry):
        for i in range(4):
            j = q * 4 + i
            r = i
            r2 = (i + 2) % 4
            gather_copy(mb[r], gb[r], gsem[r]).wait()
            compute(mb[r], db[r], gb[r])
            scat_copy(db[r], gb[r], ssem[r]).start(add=True)
            meta_copy(j + 4, mb[r], msem[r]).start()
            if i < 2:
                @pl.when(q > 0)
                def _():
                    scat_copy(db[r2], gb[r2], ssem[r2]).wait()
            else:
                scat_copy(db[r2], gb[r2], ssem[r2]).wait()
            meta_copy(j + 2, mb[r2], msem[r2]).wait()
            gather_copy(mb[r2], gb[r2], gsem[r2]).start()
        return carry

    lax.fori_loop(0, CHT // 4, quad, 0)

    scat_copy(db[2], gb[2], ssem[2]).wait()
    scat_copy(db[3], gb[3], ssem[3]).wait()
    gather_copy(mb[0], gb[0], gsem[0]).wait()
    gather_copy(mb[1], gb[1], gsem[1]).wait()
    meta_copy(CHT + 2, mb[2], msem[2]).wait()
    meta_copy(CHT + 3, mb[3], msem[3]).wait()

    plsc.subcore_barrier()
    pltpu.sync_copy(hsh.at[pl.ds(s * 640, 640)],
                    hpart.at[c, pl.ds(s * 640, 640)])


def kernel(x, prev_h, emb_rel, edge_index, edge_type, edge_time,
           weight_neighbor, attn_w, delta):
    del prev_h
    f32, i32 = jnp.float32, jnp.int32

    xp = jnp.pad(x, ((0, NP - N), (0, 0)))
    ep = jnp.pad(emb_rel, ((0, 256 - R), (0, 0)))
    npad = EP - E
    src_p = jnp.pad(edge_index[0].astype(i32), (0, npad))
    dst_p = jnp.concatenate([
        edge_index[1].astype(i32),
        N + (jnp.arange(npad, dtype=i32) % (NP - N)),
    ])
    typ_p = jnp.pad(edge_type.astype(i32), (0, npad))
    tim_p = jnp.pad(edge_time.astype(f32), (0, npad))
    src2 = src_p.reshape(NCH, CB)
    dst2 = dst_p.reshape(NCH, CB)
    typ2 = typ_p.reshape(NCH, CB)
    delta16 = jnp.broadcast_to(delta.astype(f32), (L,))
    ridx = jnp.arange(NPR, dtype=i32).reshape(5, 128)

    xa, s1o, s2o = pl.pallas_call(
        _dense_x_body,
        grid=(NP // 512,),
        in_specs=[
            pl.BlockSpec((512, D), lambda i: (i, 0)),
            pl.BlockSpec((2 * D, D), lambda i: (0, 0)),
            pl.BlockSpec((3 * D, 1), lambda i: (0, 0)),
        ],
        out_specs=[
            pl.BlockSpec((512, D), lambda i: (i, 0)),
            pl.BlockSpec((512, 1), lambda i: (i, 0)),
            pl.BlockSpec((512, 1), lambda i: (i, 0)),
        ],
        out_shape=[
            jax.ShapeDtypeStruct((NP, D), f32),
            jax.ShapeDtypeStruct((NP, 1), f32),
            jax.ShapeDtypeStruct((NP, 1), f32),
        ],
    )(xp, weight_neighbor, attn_w)

    ra, srl = pl.pallas_call(
        _dense_rel_body,
        in_specs=[
            pl.BlockSpec((256, D), lambda: (0, 0)),
            pl.BlockSpec((2 * D, D), lambda: (0, 0)),
            pl.BlockSpec((3 * D, 1), lambda: (0, 0)),
        ],
        out_specs=[
            pl.BlockSpec((256, D), lambda: (0, 0)),
            pl.BlockSpec((256, 1), lambda: (0, 0)),
        ],
        out_shape=[
            jax.ShapeDtypeStruct((256, D), f32),
            jax.ShapeDtypeStruct((256, 1), f32),
        ],
    )(ep, weight_neighbor, attn_w)

    s1p = s1o.reshape(NP)
    s2p = s2o.reshape(NP)
    srp = srl.reshape(256)
    xa2 = jnp.stack([xa[:, :DH], xa[:, DH:]])
    ra2f = jnp.concatenate([ra[:R, :DH], ra[:R, DH:]]).reshape(2 * R * DH)

    mesh = plsc.VectorSubcoreMesh(core_axis_name="c", subcore_axis_name="s",
                                  num_cores=NC, num_subcores=NS)
    pass1 = functools.partial(
        pl.kernel,
        out_type=[
            jax.ShapeDtypeStruct((EP,), f32),
            jax.ShapeDtypeStruct((NC, NPR, L), f32),
        ],
        mesh=mesh,
        scratch_types=[
            pltpu.VMEM((ET,), i32),
            pltpu.VMEM((ET,), i32),
            pltpu.VMEM((ET,), i32),
            pltpu.VMEM((ET,), f32),
            pltpu.VMEM((NP,), f32),
            pltpu.VMEM((NP,), f32),
            pltpu.VMEM((256,), f32),
            pltpu.VMEM((L,), f32),
            pltpu.VMEM((ET,), f32),
            pltpu.VMEM((NPR, L), f32),
            pltpu.VMEM((NP,), i32),
            pltpu.VMEM((5, 128), i32),
            pltpu.VMEM((40, L), f32),
            pltpu.VMEM_SHARED((NPR, L), f32),
        ],
        compiler_params=pltpu.CompilerParams(needs_layout_passes=False, use_tc_tiling_on_sc=False),
    )(_pass1_body)
    ex_all, den2 = pass1(src_p, dst_p, typ_p, tim_p, s1p, s2p, srp,
                         delta16, ridx)

    ex2 = lax.bitcast_convert_type(ex_all, i32).reshape(NCH, CB)
    meta = jnp.stack([src2, dst2, typ2, ex2], axis=1).reshape(4 * NCH, CB)
    meta = jnp.pad(meta, ((0, 16), (0, 0)))
    pass2 = functools.partial(
        pl.kernel,
        out_type=jax.ShapeDtypeStruct((NC, NP, DH), f32),
        mesh=mesh,
        scratch_types=(
            [pltpu.VMEM((4, CB), i32)] * 4
            + [pltpu.VMEM((1, CB), i32)] * 4
            + [
                pltpu.VMEM((NPR, L), f32),
                pltpu.VMEM((NPR, L), f32),
                pltpu.VMEM((R * DH,), f32),
            ]
            + [pltpu.VMEM((CB, DH), f32)] * 4
            + [pltpu.SemaphoreType.DMA] * 12
            + [pltpu.VMEM_SHARED((NP, DH), f32)]
        ),
        compiler_params=pltpu.CompilerParams(needs_layout_passes=False, use_tc_tiling_on_sc=False),
    )(_pass2_body)
    hpart = pass2(meta, den2, ra2f, xa2)

    return jnp.concatenate([hpart[0, :N], hpart[1, :N]], axis=1)


# speedup vs baseline: 1.0602x; 1.0602x over previous
import functools

import jax
import jax.numpy as jnp
from jax import lax
from jax.experimental import pallas as pl
from jax.experimental.pallas import tpu as pltpu
from jax.experimental.pallas import tpu_sc as plsc

N = 10000
D = 128
R = 200
E = 320000

L = 16
NC = 2
NS = 16
NW = NC * NS
NP = 10240
EP = 327680
ET = EP // NW
CB = 128
NCH = EP // CB
CHT = NCH // NS
NPR = NP // L
DH = D // 2


def _dense_x_body(x_ref, w_ref, aw_ref, xa_ref, s1_ref, s2_ref):
    xb = x_ref[...]
    w1 = w_ref[pl.ds(0, D), :]
    xa_ref[...] = jnp.dot(xb, w1, preferred_element_type=jnp.float32)
    a1 = aw_ref[pl.ds(0, D), :]
    a2 = aw_ref[pl.ds(D, D), :]
    s1_ref[...] = jnp.dot(xb, a1, preferred_element_type=jnp.float32)
    s2_ref[...] = jnp.dot(xb, a2, preferred_element_type=jnp.float32)


def _dense_rel_body(e_ref, w_ref, aw_ref, ra_ref, sr_ref):
    eb = e_ref[...]
    w2 = w_ref[pl.ds(D, D), :]
    ra_ref[...] = jnp.dot(eb, w2, preferred_element_type=jnp.float32)
    a3 = aw_ref[pl.ds(2 * D, D), :]
    sr_ref[...] = jnp.dot(eb, a3, preferred_element_type=jnp.float32)


def _pass1_body(srcf, dstf, typf, timf, s1h, s2h, srh, dlh, idxh,
                ex_out, den_out,
                sv, dv, tv, tm, s1r, s2r, srr, dlr, exr, denr, markr,
                idxr, zbuf, dsh):
    c = lax.axis_index("c")
    s = lax.axis_index("s")
    wid = c * NS + s
    base = wid * ET

    pltpu.sync_copy(srcf.at[pl.ds(base, ET)], sv)
    pltpu.sync_copy(dstf.at[pl.ds(base, ET)], dv)
    pltpu.sync_copy(typf.at[pl.ds(base, ET)], tv)
    pltpu.sync_copy(timf.at[pl.ds(base, ET)], tm)
    pltpu.sync_copy(s1h, s1r)
    pltpu.sync_copy(s2h, s2r)
    pltpu.sync_copy(srh, srr)
    pltpu.sync_copy(dlh, dlr)
    pltpu.sync_copy(idxh, idxr)

    zz = jnp.zeros((L,), jnp.float32)

    def zbody(i, carry):
        denr[i, pl.ds(0, L)] = zz
        return carry

    lax.fori_loop(0, NPR, zbody, 0)
    for q in range(40):
        zbuf[q, pl.ds(0, L)] = zz

    iot = lax.iota(jnp.int32, L)
    dlv = dlr[pl.ds(0, L)]

    def body(i, carry):
        sl = pl.ds(i * L, L)
        svv = sv[sl]
        dvv = dv[sl]
        tvv = tv[sl]
        tmv = tm[sl]
        a = plsc.load_gather(s1r, [svv])
        b = plsc.load_gather(s2r, [dvv])
        cc = plsc.load_gather(srr, [tvv])
        z = a + b + cc
        e = jnp.where(z >= 0.0, z, z * jnp.float32(0.01))
        ex = jnp.exp(-(tmv * dlv) * e)
        exr[sl] = ex
        dhi = lax.shift_right_logical(dvv, 4)
        dlo = lax.bitwise_and(dvv, 15)

        def one(active):
            am = active != 0
            plsc.store_scatter(markr, [dvv], iot, mask=am)
            got = plsc.load_gather(markr, [dvv], mask=am)
            win = jnp.logical_and(am, got == iot)
            plsc.addupdate_scatter(denr, [dhi, dlo], ex, mask=win)
            return jnp.where(win, 0, active)

        active = one(jnp.ones((L,), jnp.int32))
        active = lax.while_loop(lambda a_: jnp.max(a_) > 0, one, active)
        return carry

    lax.fori_loop(0, ET // L, body, 0)

    pltpu.sync_copy(exr, ex_out.at[pl.ds(base, ET)])

    pltpu.sync_copy(zbuf, dsh.at[pl.ds(s * 40, 40)])
    plsc.subcore_barrier()
    for q in range(5):
        pltpu.sync_copy(denr.at[pl.ds(q * 128, 128)], dsh.at[idxr.at[q]],
                        add=True)
    plsc.subcore_barrier()
    pltpu.sync_copy(dsh.at[pl.ds(s * 40, 40)], den_out.at[c, pl.ds(s * 40, 40)])


def _winv_body(d_ref, o_ref):
    d = d_ref[0] + d_ref[1]
    o_ref[...] = (1.0 / jnp.where(d > 0.0, d, 1.0))[None, :]


def _bcast_lane(v, l):
    idx = jnp.full((L, 1), l, jnp.int32)
    dn = lax.GatherDimensionNumbers(offset_dims=(), collapsed_slice_dims=(0,),
                                    start_index_map=(0,))
    return lax.gather(v, idx, dn, (1,),
                      mode=lax.GatherScatterMode.PROMISE_IN_BOUNDS)


def _pass2_body(metah, winvh, ra2f, xah,
                hpart,
                m0, m1, m2, m3, d0, d1, d2, d3, winvr, rar,
                g0, g1, g2, g3,
                ms0, ms1, ms2, ms3, gs0, gs1, gs2, gs3,
                ss0, ss1, ss2, ss3, hsh):
    c = lax.axis_index("c")
    s = lax.axis_index("s")
    mb = [m0, m1, m2, m3]
    db = [d0, d1, d2, d3]
    gb = [g0, g1, g2, g3]
    msem = [ms0, ms1, ms2, ms3]
    gsem = [gs0, gs1, gs2, gs3]
    ssem = [ss0, ss1, ss2, ss3]

    zz = jnp.zeros((L,), jnp.float32)

    def zb(r, carry):
        for k in range(DH // L):
            g0[r, pl.ds(k * L, L)] = zz
        return carry

    lax.fori_loop(0, CB, zb, 0)
    for q in range(5):
        pltpu.sync_copy(g0, hsh.at[pl.ds(s * 640 + q * CB, CB)])
    plsc.subcore_barrier()

    pltpu.sync_copy(winvh, winvr)
    pltpu.sync_copy(ra2f.at[pl.ds(c * (R * DH), R * DH)], rar)

    iot = lax.iota(jnp.int32, L)

    def meta_copy(j, mbuf, msem):
        return pltpu.make_async_copy(
            metah.at[pl.ds((s * CHT + j) * 4, 4)], mbuf, msem)

    def gather_copy(mbuf, gbuf, gsem):
        return pltpu.make_async_copy(xah.at[c].at[mbuf.at[0]], gbuf, gsem)

    def scat_copy(dbuf, gbuf, ssem):
        return pltpu.make_async_copy(gbuf, hsh.at[dbuf.at[0]], ssem)

    def compute(mbuf, dbuf, gbuf):
        @plsc.parallel_loop(0, CB // L)
        def grp(g):
            gl = pl.ds(g * L, L)
            dvv = mbuf[1, gl]
            dbuf[0, gl] = dvv
            tvv = mbuf[2, gl]
            exv = plsc.bitcast(mbuf[3, gl], jnp.float32)
            wv = exv * plsc.load_gather(winvr, [dvv])
            for l in range(L):
                wb = _bcast_lane(wv, l)
                tb = _bcast_lane(tvv, l)
                r = g * L + l
                for k in range(DH // L):
                    idx = tb * DH + (k * L + iot)
                    rc = plsc.load_gather(rar, [idx])
                    gv = gbuf[r, pl.ds(k * L, L)]
                    gbuf[r, pl.ds(k * L, L)] = wb * (gv + rc)

    for r in range(4):
        meta_copy(r, mb[r], msem[r]).start()
    for r in range(2):
        meta_copy(r, mb[r], msem[r]).wait()
        gather_copy(mb[r], gb[r], gsem[r]).start()

    def quad(q, carry):
        for i in range(4):
            j = q * 4 + i
            r = i
            r2 = (i + 2) % 4
            gather_copy(mb[r], gb[r], gsem[r]).wait()
            compute(mb[r], db[r], gb[r])
            scat_copy(db[r], gb[r], ssem[r]).start(add=True)
            meta_copy(j + 4, mb[r], msem[r]).start()
            if i < 2:
                @pl.when(q > 0)
                def _():
                    scat_copy(db[r2], gb[r2], ssem[r2]).wait()
            else:
                scat_copy(db[r2], gb[r2], ssem[r2]).wait()
            meta_copy(j + 2, mb[r2], msem[r2]).wait()
            gather_copy(mb[r2], gb[r2], gsem[r2]).start()
        return carry

    lax.fori_loop(0, CHT // 4, quad, 0)

    scat_copy(db[2], gb[2], ssem[2]).wait()
    scat_copy(db[3], gb[3], ssem[3]).wait()
    gather_copy(mb[0], gb[0], gsem[0]).wait()
    gather_copy(mb[1], gb[1], gsem[1]).wait()
    meta_copy(CHT + 2, mb[2], msem[2]).wait()
    meta_copy(CHT + 3, mb[3], msem[3]).wait()

    plsc.subcore_barrier()
    pltpu.sync_copy(hsh.at[pl.ds(s * 640, 640)],
                    hpart.at[c, pl.ds(s * 640, 640)])


def kernel(x, prev_h, emb_rel, edge_index, edge_type, edge_time,
           weight_neighbor, attn_w, delta):
    del prev_h
    f32, i32 = jnp.float32, jnp.int32

    xp = jnp.pad(x, ((0, NP - N), (0, 0)))
    ep = jnp.pad(emb_rel, ((0, 256 - R), (0, 0)))
    npad = EP - E
    src_p = jnp.pad(edge_index[0].astype(i32), (0, npad))
    dst_p = jnp.concatenate([
        edge_index[1].astype(i32),
        N + (jnp.arange(npad, dtype=i32) % (NP - N)),
    ])
    typ_p = jnp.pad(edge_type.astype(i32), (0, npad))
    tim_p = jnp.pad(edge_time.astype(f32), (0, npad))
    src2 = src_p.reshape(NCH, CB)
    dst2 = dst_p.reshape(NCH, CB)
    typ2 = typ_p.reshape(NCH, CB)
    delta16 = jnp.broadcast_to(delta.astype(f32), (L,))
    ridx = jnp.arange(NPR, dtype=i32).reshape(5, 128)

    xa, s1o, s2o = pl.pallas_call(
        _dense_x_body,
        grid=(NP // 512,),
        in_specs=[
            pl.BlockSpec((512, D), lambda i: (i, 0)),
            pl.BlockSpec((2 * D, D), lambda i: (0, 0)),
            pl.BlockSpec((3 * D, 1), lambda i: (0, 0)),
        ],
        out_specs=[
            pl.BlockSpec((512, D), lambda i: (i, 0)),
            pl.BlockSpec((512, 1), lambda i: (i, 0)),
            pl.BlockSpec((512, 1), lambda i: (i, 0)),
        ],
        out_shape=[
            jax.ShapeDtypeStruct((NP, D), f32),
            jax.ShapeDtypeStruct((NP, 1), f32),
            jax.ShapeDtypeStruct((NP, 1), f32),
        ],
    )(xp, weight_neighbor, attn_w)

    ra, srl = pl.pallas_call(
        _dense_rel_body,
        in_specs=[
            pl.BlockSpec((256, D), lambda: (0, 0)),
            pl.BlockSpec((2 * D, D), lambda: (0, 0)),
            pl.BlockSpec((3 * D, 1), lambda: (0, 0)),
        ],
        out_specs=[
            pl.BlockSpec((256, D), lambda: (0, 0)),
            pl.BlockSpec((256, 1), lambda: (0, 0)),
        ],
        out_shape=[
            jax.ShapeDtypeStruct((256, D), f32),
            jax.ShapeDtypeStruct((256, 1), f32),
        ],
    )(ep, weight_neighbor, attn_w)

    s1p = s1o.reshape(NP)
    s2p = s2o.reshape(NP)
    srp = srl.reshape(256)
    xa2 = jnp.stack([xa[:, :DH], xa[:, DH:]])
    ra2f = jnp.concatenate([ra[:R, :DH], ra[:R, DH:]]).reshape(2 * R * DH)

    mesh = plsc.VectorSubcoreMesh(core_axis_name="c", subcore_axis_name="s",
                                  num_cores=NC, num_subcores=NS)
    pass1 = functools.partial(
        pl.kernel,
        out_type=[
            jax.ShapeDtypeStruct((EP,), f32),
            jax.ShapeDtypeStruct((NC, NPR, L), f32),
        ],
        mesh=mesh,
        scratch_types=[
            pltpu.VMEM((ET,), i32),
            pltpu.VMEM((ET,), i32),
            pltpu.VMEM((ET,), i32),
            pltpu.VMEM((ET,), f32),
            pltpu.VMEM((NP,), f32),
            pltpu.VMEM((NP,), f32),
            pltpu.VMEM((256,), f32),
            pltpu.VMEM((L,), f32),
            pltpu.VMEM((ET,), f32),
            pltpu.VMEM((NPR, L), f32),
            pltpu.VMEM((NP,), i32),
            pltpu.VMEM((5, 128), i32),
            pltpu.VMEM((40, L), f32),
            pltpu.VMEM_SHARED((NPR, L), f32),
        ],
        compiler_params=pltpu.CompilerParams(needs_layout_passes=False, use_tc_tiling_on_sc=False),
    )(_pass1_body)
    ex_all, den2 = pass1(src_p, dst_p, typ_p, tim_p, s1p, s2p, srp,
                         delta16, ridx)

    winv = pl.pallas_call(
        _winv_body,
        grid=(4,),
        in_specs=[pl.BlockSpec((NC, NP // 4), lambda i: (0, i))],
        out_specs=pl.BlockSpec((1, NP // 4), lambda i: (0, i)),
        out_shape=jax.ShapeDtypeStruct((1, NP), f32),
    )(den2.reshape(NC, NP)).reshape(NP)

    ex2 = lax.bitcast_convert_type(ex_all, i32).reshape(NCH, CB)
    meta = jnp.stack([src2, dst2, typ2, ex2], axis=1).reshape(4 * NCH, CB)
    meta = jnp.pad(meta, ((0, 16), (0, 0)))
    pass2 = functools.partial(
        pl.kernel,
        out_type=jax.ShapeDtypeStruct((NC, NP, DH), f32),
        mesh=mesh,
        scratch_types=(
            [pltpu.VMEM((4, CB), i32)] * 4
            + [pltpu.VMEM((1, CB), i32)] * 4
            + [
                pltpu.VMEM((NP,), f32),
                pltpu.VMEM((R * DH,), f32),
            ]
            + [pltpu.VMEM((CB, DH), f32)] * 4
            + [pltpu.SemaphoreType.DMA] * 12
            + [pltpu.VMEM_SHARED((NP, DH), f32)]
        ),
        compiler_params=pltpu.CompilerParams(needs_layout_passes=False, use_tc_tiling_on_sc=False),
    )(_pass2_body)
    hpart = pass2(meta, winv, ra2f, xa2)

    return jnp.concatenate([hpart[0, :N], hpart[1, :N]], axis=1)


# speedup vs baseline: 1.3143x; 1.2396x over previous
import functools

import jax
import jax.numpy as jnp
from jax import lax
from jax.experimental import pallas as pl
from jax.experimental.pallas import tpu as pltpu
from jax.experimental.pallas import tpu_sc as plsc

N = 10000
D = 128
R = 200
E = 320000

L = 16
NC = 2
NS = 16
NW = NC * NS
NP = 10240
EP = 327680
ET = EP // NW
CB = 128
NCH = EP // CB
CHT = NCH // NS
NPR = NP // L
DH = D // 2


def _dense_x_body(x_ref, w_ref, aw_ref, xa_ref, s1_ref, s2_ref):
    xb = x_ref[...]
    w1 = w_ref[pl.ds(0, D), :]
    xa_ref[...] = jnp.dot(xb, w1, preferred_element_type=jnp.float32)
    a1 = aw_ref[pl.ds(0, D), :]
    a2 = aw_ref[pl.ds(D, D), :]
    s1_ref[...] = jnp.dot(xb, a1, preferred_element_type=jnp.float32)
    s2_ref[...] = jnp.dot(xb, a2, preferred_element_type=jnp.float32)


def _dense_rel_body(e_ref, w_ref, aw_ref, ra_ref, sr_ref):
    eb = e_ref[...]
    w2 = w_ref[pl.ds(D, D), :]
    ra_ref[...] = jnp.dot(eb, w2, preferred_element_type=jnp.float32)
    a3 = aw_ref[pl.ds(2 * D, D), :]
    sr_ref[...] = jnp.dot(eb, a3, preferred_element_type=jnp.float32)


def _pass1_body(srcf, dstf, typf, timf, s1h, s2h, srh, dlh, idxh,
                ex_out, den_out,
                sv, dv, tv, tm, s1r, s2r, srr, dlr, exr, denr, markr,
                idxr, zbuf, dsh):
    c = lax.axis_index("c")
    s = lax.axis_index("s")
    wid = c * NS + s
    base = wid * ET

    pltpu.sync_copy(srcf.at[pl.ds(base, ET)], sv)
    pltpu.sync_copy(dstf.at[pl.ds(base, ET)], dv)
    pltpu.sync_copy(typf.at[pl.ds(base, ET)], tv)
    pltpu.sync_copy(timf.at[pl.ds(base, ET)], tm)
    pltpu.sync_copy(s1h, s1r)
    pltpu.sync_copy(s2h, s2r)
    pltpu.sync_copy(srh, srr)
    pltpu.sync_copy(dlh, dlr)
    pltpu.sync_copy(idxh, idxr)

    zz = jnp.zeros((L,), jnp.float32)

    def zbody(i, carry):
        denr[i, pl.ds(0, L)] = zz
        return carry

    lax.fori_loop(0, NPR, zbody, 0)
    for q in range(40):
        zbuf[q, pl.ds(0, L)] = zz

    iot = lax.iota(jnp.int32, L)
    dlv = dlr[pl.ds(0, L)]

    def body(i, carry):
        sl = pl.ds(i * L, L)
        svv = sv[sl]
        dvv = dv[sl]
        tvv = tv[sl]
        tmv = tm[sl]
        a = plsc.load_gather(s1r, [svv])
        b = plsc.load_gather(s2r, [dvv])
        cc = plsc.load_gather(srr, [tvv])
        z = a + b + cc
        e = jnp.where(z >= 0.0, z, z * jnp.float32(0.01))
        ex = jnp.exp(-(tmv * dlv) * e)
        exr[sl] = ex
        dhi = lax.shift_right_logical(dvv, 4)
        dlo = lax.bitwise_and(dvv, 15)

        def one(active):
            am = active != 0
            plsc.store_scatter(markr, [dvv], iot, mask=am)
            got = plsc.load_gather(markr, [dvv], mask=am)
            win = jnp.logical_and(am, got == iot)
            plsc.addupdate_scatter(denr, [dhi, dlo], ex, mask=win)
            return jnp.where(win, 0, active)

        active = one(jnp.ones((L,), jnp.int32))
        active = lax.while_loop(lambda a_: jnp.max(a_) > 0, one, active)
        return carry

    lax.fori_loop(0, ET // L, body, 0)

    pltpu.sync_copy(exr, ex_out.at[pl.ds(base, ET)])

    pltpu.sync_copy(zbuf, dsh.at[pl.ds(s * 40, 40)])
    plsc.subcore_barrier()
    for q in range(5):
        pltpu.sync_copy(denr.at[pl.ds(q * 128, 128)], dsh.at[idxr.at[q]],
                        add=True)
    plsc.subcore_barrier()
    pltpu.sync_copy(dsh.at[pl.ds(s * 40, 40)], den_out.at[c, pl.ds(s * 40, 40)])


def _winv_body(d_ref, o_ref):
    d = d_ref[0] + d_ref[1]
    o_ref[...] = (1.0 / jnp.where(d > 0.0, d, 1.0))[None, :]


def _bcast_lane(v, l):
    idx = jnp.full((L, 1), l, jnp.int32)
    dn = lax.GatherDimensionNumbers(offset_dims=(), collapsed_slice_dims=(0,),
                                    start_index_map=(0,))
    return lax.gather(v, idx, dn, (1,),
                      mode=lax.GatherScatterMode.PROMISE_IN_BOUNDS)


def _pass2_body(metah, winvh, ra2f, xah,
                hpart,
                m0, m1, m2, m3, d0, d1, d2, d3, winvr, rar,
                g0, g1, g2, g3, o0, o1, o2, o3,
                ms0, ms1, ms2, ms3, gs0, gs1, gs2, gs3,
                ss0, ss1, ss2, ss3, hsh):
    c = lax.axis_index("c")
    s = lax.axis_index("s")
    mb = [m0, m1, m2, m3]
    db = [d0, d1, d2, d3]
    gb = [g0, g1, g2, g3]
    ob = [o0, o1, o2, o3]
    msem = [ms0, ms1, ms2, ms3]
    gsem = [gs0, gs1, gs2, gs3]
    ssem = [ss0, ss1, ss2, ss3]

    zz = jnp.zeros((L,), jnp.float32)

    def zb(r, carry):
        for k in range(DH // L):
            o0[r, pl.ds(k * L, L)] = zz
        return carry

    lax.fori_loop(0, CB, zb, 0)
    for q in range(5):
        pltpu.sync_copy(o0, hsh.at[pl.ds(s * 640 + q * CB, CB)])
    plsc.subcore_barrier()

    pltpu.sync_copy(winvh, winvr)
    pltpu.sync_copy(ra2f.at[pl.ds(c * (R * DH // 2), R * DH // 2)], rar)

    iot = lax.iota(jnp.int32, L)

    def meta_copy(j, mbuf, msem):
        return pltpu.make_async_copy(
            metah.at[pl.ds((s * CHT + j) * 4, 4)], mbuf, msem)

    def gather_copy(mbuf, gbuf, gsem):
        return pltpu.make_async_copy(xah.at[c].at[mbuf.at[0]], gbuf, gsem)

    def scat_copy(dbuf, obuf, ssem):
        return pltpu.make_async_copy(obuf, hsh.at[dbuf.at[0]], ssem)

    def compute(mbuf, dbuf, gbuf, obuf):
        @plsc.parallel_loop(0, CB // L)
        def grp(g):
            gl = pl.ds(g * L, L)
            dvv = mbuf[1, gl]
            dbuf[0, gl] = dvv
            tvv = mbuf[2, gl]
            exv = plsc.bitcast(mbuf[3, gl], jnp.float32)
            wv = exv * plsc.load_gather(winvr, [dvv])
            for l in range(L):
                wb = _bcast_lane(wv, l)
                tb = _bcast_lane(tvv, l) * (DH // 2)
                r = g * L + l
                for k in range(2):
                    gi = gbuf[r, pl.ds(k * L, L)]
                    glo, ghi = plsc.unpack(
                        plsc.bitcast(gi, jnp.bfloat16),
                        format=plsc.PackFormat.INTERLEAVED)
                    ri = plsc.load_gather(rar, [tb + (k * L + iot)])
                    rlo, rhi = plsc.unpack(
                        plsc.bitcast(ri, jnp.bfloat16),
                        format=plsc.PackFormat.INTERLEAVED)
                    obuf[r, pl.ds(k * 2 * L, L)] = wb * (glo + rlo)
                    obuf[r, pl.ds(k * 2 * L + L, L)] = wb * (ghi + rhi)

    for r in range(4):
        meta_copy(r, mb[r], msem[r]).start()
    for r in range(2):
        meta_copy(r, mb[r], msem[r]).wait()
        gather_copy(mb[r], gb[r], gsem[r]).start()

    def quad(q, carry):
        for i in range(4):
            j = q * 4 + i
            r = i
            r2 = (i + 2) % 4
            gather_copy(mb[r], gb[r], gsem[r]).wait()
            compute(mb[r], db[r], gb[r], ob[r])
            scat_copy(db[r], ob[r], ssem[r]).start(add=True)
            meta_copy(j + 4, mb[r], msem[r]).start()
            if i < 2:
                @pl.when(q > 0)
                def _():
                    scat_copy(db[r2], ob[r2], ssem[r2]).wait()
            else:
                scat_copy(db[r2], ob[r2], ssem[r2]).wait()
            meta_copy(j + 2, mb[r2], msem[r2]).wait()
            gather_copy(mb[r2], gb[r2], gsem[r2]).start()
        return carry

    lax.fori_loop(0, CHT // 4, quad, 0)

    scat_copy(db[2], ob[2], ssem[2]).wait()
    scat_copy(db[3], ob[3], ssem[3]).wait()
    gather_copy(mb[0], gb[0], gsem[0]).wait()
    gather_copy(mb[1], gb[1], gsem[1]).wait()
    meta_copy(CHT + 2, mb[2], msem[2]).wait()
    meta_copy(CHT + 3, mb[3], msem[3]).wait()

    plsc.subcore_barrier()
    pltpu.sync_copy(hsh.at[pl.ds(s * 640, 640)],
                    hpart.at[c, pl.ds(s * 640, 640)])


def kernel(x, prev_h, emb_rel, edge_index, edge_type, edge_time,
           weight_neighbor, attn_w, delta):
    del prev_h
    f32, i32 = jnp.float32, jnp.int32

    xp = jnp.pad(x, ((0, NP - N), (0, 0)))
    ep = jnp.pad(emb_rel, ((0, 256 - R), (0, 0)))
    npad = EP - E
    src_p = jnp.pad(edge_index[0].astype(i32), (0, npad))
    dst_p = jnp.concatenate([
        edge_index[1].astype(i32),
        N + (jnp.arange(npad, dtype=i32) % (NP - N)),
    ])
    typ_p = jnp.pad(edge_type.astype(i32), (0, npad))
    tim_p = jnp.pad(edge_time.astype(f32), (0, npad))
    src2 = src_p.reshape(NCH, CB)
    dst2 = dst_p.reshape(NCH, CB)
    typ2 = typ_p.reshape(NCH, CB)
    delta16 = jnp.broadcast_to(delta.astype(f32), (L,))
    ridx = jnp.arange(NPR, dtype=i32).reshape(5, 128)

    xa, s1o, s2o = pl.pallas_call(
        _dense_x_body,
        grid=(NP // 512,),
        in_specs=[
            pl.BlockSpec((512, D), lambda i: (i, 0)),
            pl.BlockSpec((2 * D, D), lambda i: (0, 0)),
            pl.BlockSpec((3 * D, 1), lambda i: (0, 0)),
        ],
        out_specs=[
            pl.BlockSpec((512, D), lambda i: (i, 0)),
            pl.BlockSpec((512, 1), lambda i: (i, 0)),
            pl.BlockSpec((512, 1), lambda i: (i, 0)),
        ],
        out_shape=[
            jax.ShapeDtypeStruct((NP, D), f32),
            jax.ShapeDtypeStruct((NP, 1), f32),
            jax.ShapeDtypeStruct((NP, 1), f32),
        ],
    )(xp, weight_neighbor, attn_w)

    ra, srl = pl.pallas_call(
        _dense_rel_body,
        in_specs=[
            pl.BlockSpec((256, D), lambda: (0, 0)),
            pl.BlockSpec((2 * D, D), lambda: (0, 0)),
            pl.BlockSpec((3 * D, 1), lambda: (0, 0)),
        ],
        out_specs=[
            pl.BlockSpec((256, D), lambda: (0, 0)),
            pl.BlockSpec((256, 1), lambda: (0, 0)),
        ],
        out_shape=[
            jax.ShapeDtypeStruct((256, D), f32),
            jax.ShapeDtypeStruct((256, 1), f32),
        ],
    )(ep, weight_neighbor, attn_w)

    s1p = s1o.reshape(NP)
    s2p = s2o.reshape(NP)
    srp = srl.reshape(256)
    perm = []
    for b in range(2):
        for i in range(L):
            perm += [b * 2 * L + i, b * 2 * L + L + i]
    bf16 = jnp.bfloat16
    xa2 = jnp.stack([xa[:, :DH], xa[:, DH:]])[:, :, perm].astype(bf16)
    xa2 = lax.bitcast_convert_type(xa2.reshape(NC, NP, DH // 2, 2), i32)
    ra2 = jnp.stack([ra[:R, :DH], ra[:R, DH:]])[:, :, perm].astype(bf16)
    ra2f = lax.bitcast_convert_type(
        ra2.reshape(NC, R, DH // 2, 2), i32).reshape(NC * R * (DH // 2))

    mesh = plsc.VectorSubcoreMesh(core_axis_name="c", subcore_axis_name="s",
                                  num_cores=NC, num_subcores=NS)
    pass1 = functools.partial(
        pl.kernel,
        out_type=[
            jax.ShapeDtypeStruct((EP,), f32),
            jax.ShapeDtypeStruct((NC, NPR, L), f32),
        ],
        mesh=mesh,
        scratch_types=[
            pltpu.VMEM((ET,), i32),
            pltpu.VMEM((ET,), i32),
            pltpu.VMEM((ET,), i32),
            pltpu.VMEM((ET,), f32),
            pltpu.VMEM((NP,), f32),
            pltpu.VMEM((NP,), f32),
            pltpu.VMEM((256,), f32),
            pltpu.VMEM((L,), f32),
            pltpu.VMEM((ET,), f32),
            pltpu.VMEM((NPR, L), f32),
            pltpu.VMEM((NP,), i32),
            pltpu.VMEM((5, 128), i32),
            pltpu.VMEM((40, L), f32),
            pltpu.VMEM_SHARED((NPR, L), f32),
        ],
        compiler_params=pltpu.CompilerParams(needs_layout_passes=False, use_tc_tiling_on_sc=False),
    )(_pass1_body)
    ex_all, den2 = pass1(src_p, dst_p, typ_p, tim_p, s1p, s2p, srp,
                         delta16, ridx)

    winv = pl.pallas_call(
        _winv_body,
        grid=(4,),
        in_specs=[pl.BlockSpec((NC, NP // 4), lambda i: (0, i))],
        out_specs=pl.BlockSpec((1, NP // 4), lambda i: (0, i)),
        out_shape=jax.ShapeDtypeStruct((1, NP), f32),
    )(den2.reshape(NC, NP)).reshape(NP)

    ex2 = lax.bitcast_convert_type(ex_all, i32).reshape(NCH, CB)
    meta = jnp.stack([src2, dst2, typ2, ex2], axis=1).reshape(4 * NCH, CB)
    meta = jnp.pad(meta, ((0, 16), (0, 0)))
    pass2 = functools.partial(
        pl.kernel,
        out_type=jax.ShapeDtypeStruct((NC, NP, DH), f32),
        mesh=mesh,
        scratch_types=(
            [pltpu.VMEM((4, CB), i32)] * 4
            + [pltpu.VMEM((1, CB), i32)] * 4
            + [
                pltpu.VMEM((NP,), f32),
                pltpu.VMEM((R * DH // 2,), i32),
            ]
            + [pltpu.VMEM((CB, DH // 2), i32)] * 4
            + [pltpu.VMEM((CB, DH), f32)] * 4
            + [pltpu.SemaphoreType.DMA] * 12
            + [pltpu.VMEM_SHARED((NP, DH), f32)]
        ),
        compiler_params=pltpu.CompilerParams(needs_layout_passes=False, use_tc_tiling_on_sc=False),
    )(_pass2_body)
    hpart = pass2(meta, winv, ra2f, xa2)

    return jnp.concatenate([hpart[0, :N], hpart[1, :N]], axis=1)


# speedup vs baseline: 1.3366x; 1.0169x over previous
import functools

import jax
import jax.numpy as jnp
from jax import lax
from jax.experimental import pallas as pl
from jax.experimental.pallas import tpu as pltpu
from jax.experimental.pallas import tpu_sc as plsc

N = 10000
D = 128
R = 200
E = 320000

L = 16
NC = 2
NS = 16
NW = NC * NS
NP = 10240
EP = 327680
ET = EP // NW
CB = 128
NCH = EP // CB
CHT = NCH // NS
NPR = NP // L
DH = D // 2


def _dense_x_body(x_ref, w_ref, aw_ref, xa_ref, s1_ref, s2_ref):
    xb = x_ref[...]
    w1 = w_ref[pl.ds(0, D), :]
    xa_ref[...] = jnp.dot(xb, w1, preferred_element_type=jnp.float32)
    a1 = aw_ref[pl.ds(0, D), :]
    a2 = aw_ref[pl.ds(D, D), :]
    s1_ref[...] = jnp.dot(xb, a1, preferred_element_type=jnp.float32)
    s2_ref[...] = jnp.dot(xb, a2, preferred_element_type=jnp.float32)


def _dense_rel_body(e_ref, w_ref, aw_ref, ra_ref, sr_ref):
    eb = e_ref[...]
    w2 = w_ref[pl.ds(D, D), :]
    ra_ref[...] = jnp.dot(eb, w2, preferred_element_type=jnp.float32)
    a3 = aw_ref[pl.ds(2 * D, D), :]
    sr_ref[...] = jnp.dot(eb, a3, preferred_element_type=jnp.float32)


def _pass1_body(srcf, dstf, typf, timf, s1h, s2h, srh, dlh, idxh,
                ex_out, den_out,
                sv, dv, tv, tm, s1r, s2r, srr, dlr, exr, denr, markr,
                idxr, zbuf, dsh):
    c = lax.axis_index("c")
    s = lax.axis_index("s")
    wid = c * NS + s
    base = wid * ET

    pltpu.sync_copy(srcf.at[pl.ds(base, ET)], sv)
    pltpu.sync_copy(dstf.at[pl.ds(base, ET)], dv)
    pltpu.sync_copy(typf.at[pl.ds(base, ET)], tv)
    pltpu.sync_copy(timf.at[pl.ds(base, ET)], tm)
    pltpu.sync_copy(s1h, s1r)
    pltpu.sync_copy(s2h, s2r)
    pltpu.sync_copy(srh, srr)
    pltpu.sync_copy(dlh, dlr)
    pltpu.sync_copy(idxh, idxr)

    zz = jnp.zeros((L,), jnp.float32)

    def zbody(i, carry):
        denr[i, pl.ds(0, L)] = zz
        return carry

    lax.fori_loop(0, NPR, zbody, 0)
    for q in range(40):
        zbuf[q, pl.ds(0, L)] = zz

    iot = lax.iota(jnp.int32, L)
    dlv = dlr[pl.ds(0, L)]

    def body(i, carry):
        sl = pl.ds(i * L, L)
        svv = sv[sl]
        dvv = dv[sl]
        tvv = tv[sl]
        tmv = tm[sl]
        a = plsc.load_gather(s1r, [svv])
        b = plsc.load_gather(s2r, [dvv])
        cc = plsc.load_gather(srr, [tvv])
        z = a + b + cc
        e = jnp.where(z >= 0.0, z, z * jnp.float32(0.01))
        ex = jnp.exp(-(tmv * dlv) * e)
        exr[sl] = ex
        dhi = lax.shift_right_logical(dvv, 4)
        dlo = lax.bitwise_and(dvv, 15)

        def one(active):
            am = active != 0
            plsc.store_scatter(markr, [dvv], iot, mask=am)
            got = plsc.load_gather(markr, [dvv], mask=am)
            win = jnp.logical_and(am, got == iot)
            plsc.addupdate_scatter(denr, [dhi, dlo], ex, mask=win)
            return jnp.where(win, 0, active)

        active = one(jnp.ones((L,), jnp.int32))
        active = lax.while_loop(lambda a_: jnp.max(a_) > 0, one, active)
        return carry

    lax.fori_loop(0, ET // L, body, 0)

    pltpu.sync_copy(exr, ex_out.at[pl.ds(base, ET)])

    pltpu.sync_copy(zbuf, dsh.at[pl.ds(s * 40, 40)])
    plsc.subcore_barrier()
    for q in range(5):
        pltpu.sync_copy(denr.at[pl.ds(q * 128, 128)], dsh.at[idxr.at[q]],
                        add=True)
    plsc.subcore_barrier()
    pltpu.sync_copy(dsh.at[pl.ds(s * 40, 40)], den_out.at[c, pl.ds(s * 40, 40)])


def _winv_body(d_ref, o_ref):
    d = d_ref[0] + d_ref[1]
    o_ref[...] = (1.0 / jnp.where(d > 0.0, d, 1.0))[None, :]


def _bcast_lane(v, l):
    idx = jnp.full((L, 1), l, jnp.int32)
    dn = lax.GatherDimensionNumbers(offset_dims=(), collapsed_slice_dims=(0,),
                                    start_index_map=(0,))
    return lax.gather(v, idx, dn, (1,),
                      mode=lax.GatherScatterMode.PROMISE_IN_BOUNDS)


def _pass2_body(metah, winvh, ra2f, xah,
                hpart,
                m0, m1, m2, m3, d0, d1, d2, d3, winvr, rar,
                g0, g1, g2, g3, o0, o1, o2, o3,
                ms0, ms1, ms2, ms3, gs0, gs1, gs2, gs3,
                ss0, ss1, ss2, ss3, hsh):
    c = lax.axis_index("c")
    s = lax.axis_index("s")
    mb = [m0, m1, m2, m3]
    db = [d0, d1, d2, d3]
    gb = [g0, g1, g2, g3]
    ob = [o0, o1, o2, o3]
    msem = [ms0, ms1, ms2, ms3]
    gsem = [gs0, gs1, gs2, gs3]
    ssem = [ss0, ss1, ss2, ss3]

    zz = jnp.zeros((L,), jnp.float32)

    def zb(r, carry):
        for k in range(DH // L):
            o0[r, pl.ds(k * L, L)] = zz
        return carry

    lax.fori_loop(0, CB, zb, 0)
    for q in range(5):
        pltpu.sync_copy(o0, hsh.at[pl.ds(s * 640 + q * CB, CB)])
    plsc.subcore_barrier()

    pltpu.sync_copy(winvh, winvr)
    pltpu.sync_copy(ra2f.at[pl.ds(c * (R * DH // 2), R * DH // 2)], rar)

    iot = lax.iota(jnp.int32, L)

    def meta_copy(j, mbuf, msem):
        return pltpu.make_async_copy(
            metah.at[pl.ds((s * CHT + j) * 4, 4)], mbuf, msem)

    def gather_copy(mbuf, gbuf, gsem):
        return pltpu.make_async_copy(xah.at[c].at[mbuf.at[0]], gbuf, gsem)

    def scat_copy(dbuf, obuf, ssem):
        return pltpu.make_async_copy(obuf, hsh.at[dbuf.at[0]], ssem)

    def compute(mbuf, dbuf, gbuf, obuf):
        @plsc.parallel_loop(0, CB // L)
        def grp(g):
            gl = pl.ds(g * L, L)
            dvv = mbuf[1, gl]
            dbuf[0, gl] = dvv
            tvv = mbuf[2, gl]
            exv = plsc.bitcast(mbuf[3, gl], jnp.float32)
            wv = exv * plsc.load_gather(winvr, [dvv])
            for l in range(L):
                wb = _bcast_lane(wv, l)
                tb = _bcast_lane(tvv, l) * (DH // 2)
                r = g * L + l
                for k in range(2):
                    gi = gbuf[r, pl.ds(k * L, L)]
                    glo, ghi = plsc.unpack(
                        plsc.bitcast(gi, jnp.bfloat16),
                        format=plsc.PackFormat.INTERLEAVED)
                    ri = plsc.load_gather(rar, [tb + (k * L + iot)])
                    rlo, rhi = plsc.unpack(
                        plsc.bitcast(ri, jnp.bfloat16),
                        format=plsc.PackFormat.INTERLEAVED)
                    obuf[r, pl.ds(k * 2 * L, L)] = wb * (glo + rlo)
                    obuf[r, pl.ds(k * 2 * L + L, L)] = wb * (ghi + rhi)

    for r in range(4):
        meta_copy(r, mb[r], msem[r]).start()
    for r in range(3):
        meta_copy(r, mb[r], msem[r]).wait()
        gather_copy(mb[r], gb[r], gsem[r]).start()

    def quad(q, carry):
        for i in range(4):
            j = q * 4 + i
            r = i
            r3 = (i + 3) % 4
            gather_copy(mb[r], gb[r], gsem[r]).wait()

            @pl.when(q > 0)
            def _():
                scat_copy(db[r], ob[r], ssem[r]).wait()
            compute(mb[r], db[r], gb[r], ob[r])
            scat_copy(db[r], ob[r], ssem[r]).start(add=True)
            meta_copy(j + 4, mb[r], msem[r]).start()
            meta_copy(j + 3, mb[r3], msem[r3]).wait()
            gather_copy(mb[r3], gb[r3], gsem[r3]).start()
        return carry

    lax.fori_loop(0, CHT // 4, quad, 0)

    for r in range(4):
        scat_copy(db[r], ob[r], ssem[r]).wait()
    for r in range(3):
        gather_copy(mb[r], gb[r], gsem[r]).wait()
    meta_copy(CHT + 3, mb[3], msem[3]).wait()

    plsc.subcore_barrier()
    pltpu.sync_copy(hsh.at[pl.ds(s * 640, 640)],
                    hpart.at[c, pl.ds(s * 640, 640)])


def kernel(x, prev_h, emb_rel, edge_index, edge_type, edge_time,
           weight_neighbor, attn_w, delta):
    del prev_h
    f32, i32 = jnp.float32, jnp.int32

    xp = jnp.pad(x, ((0, NP - N), (0, 0)))
    ep = jnp.pad(emb_rel, ((0, 256 - R), (0, 0)))
    npad = EP - E
    src_p = jnp.pad(edge_index[0].astype(i32), (0, npad))
    dst_p = jnp.concatenate([
        edge_index[1].astype(i32),
        N + (jnp.arange(npad, dtype=i32) % (NP - N)),
    ])
    typ_p = jnp.pad(edge_type.astype(i32), (0, npad))
    tim_p = jnp.pad(edge_time.astype(f32), (0, npad))
    src2 = src_p.reshape(NCH, CB)
    dst2 = dst_p.reshape(NCH, CB)
    typ2 = typ_p.reshape(NCH, CB)
    delta16 = jnp.broadcast_to(delta.astype(f32), (L,))
    ridx = jnp.arange(NPR, dtype=i32).reshape(5, 128)

    xa, s1o, s2o = pl.pallas_call(
        _dense_x_body,
        grid=(NP // 512,),
        in_specs=[
            pl.BlockSpec((512, D), lambda i: (i, 0)),
            pl.BlockSpec((2 * D, D), lambda i: (0, 0)),
            pl.BlockSpec((3 * D, 1), lambda i: (0, 0)),
        ],
        out_specs=[
            pl.BlockSpec((512, D), lambda i: (i, 0)),
            pl.BlockSpec((512, 1), lambda i: (i, 0)),
            pl.BlockSpec((512, 1), lambda i: (i, 0)),
        ],
        out_shape=[
            jax.ShapeDtypeStruct((NP, D), f32),
            jax.ShapeDtypeStruct((NP, 1), f32),
            jax.ShapeDtypeStruct((NP, 1), f32),
        ],
    )(xp, weight_neighbor, attn_w)

    ra, srl = pl.pallas_call(
        _dense_rel_body,
        in_specs=[
            pl.BlockSpec((256, D), lambda: (0, 0)),
            pl.BlockSpec((2 * D, D), lambda: (0, 0)),
            pl.BlockSpec((3 * D, 1), lambda: (0, 0)),
        ],
        out_specs=[
            pl.BlockSpec((256, D), lambda: (0, 0)),
            pl.BlockSpec((256, 1), lambda: (0, 0)),
        ],
        out_shape=[
            jax.ShapeDtypeStruct((256, D), f32),
            jax.ShapeDtypeStruct((256, 1), f32),
        ],
    )(ep, weight_neighbor, attn_w)

    s1p = s1o.reshape(NP)
    s2p = s2o.reshape(NP)
    srp = srl.reshape(256)
    perm = []
    for b in range(2):
        for i in range(L):
            perm += [b * 2 * L + i, b * 2 * L + L + i]
    bf16 = jnp.bfloat16
    xa2 = jnp.stack([xa[:, :DH], xa[:, DH:]])[:, :, perm].astype(bf16)
    xa2 = lax.bitcast_convert_type(xa2.reshape(NC, NP, DH // 2, 2), i32)
    ra2 = jnp.stack([ra[:R, :DH], ra[:R, DH:]])[:, :, perm].astype(bf16)
    ra2f = lax.bitcast_convert_type(
        ra2.reshape(NC, R, DH // 2, 2), i32).reshape(NC * R * (DH // 2))

    mesh = plsc.VectorSubcoreMesh(core_axis_name="c", subcore_axis_name="s",
                                  num_cores=NC, num_subcores=NS)
    pass1 = functools.partial(
        pl.kernel,
        out_type=[
            jax.ShapeDtypeStruct((EP,), f32),
            jax.ShapeDtypeStruct((NC, NPR, L), f32),
        ],
        mesh=mesh,
        scratch_types=[
            pltpu.VMEM((ET,), i32),
            pltpu.VMEM((ET,), i32),
            pltpu.VMEM((ET,), i32),
            pltpu.VMEM((ET,), f32),
            pltpu.VMEM((NP,), f32),
            pltpu.VMEM((NP,), f32),
            pltpu.VMEM((256,), f32),
            pltpu.VMEM((L,), f32),
            pltpu.VMEM((ET,), f32),
            pltpu.VMEM((NPR, L), f32),
            pltpu.VMEM((NP,), i32),
            pltpu.VMEM((5, 128), i32),
            pltpu.VMEM((40, L), f32),
            pltpu.VMEM_SHARED((NPR, L), f32),
        ],
        compiler_params=pltpu.CompilerParams(needs_layout_passes=False, use_tc_tiling_on_sc=False),
    )(_pass1_body)
    ex_all, den2 = pass1(src_p, dst_p, typ_p, tim_p, s1p, s2p, srp,
                         delta16, ridx)

    winv = pl.pallas_call(
        _winv_body,
        grid=(4,),
        in_specs=[pl.BlockSpec((NC, NP // 4), lambda i: (0, i))],
        out_specs=pl.BlockSpec((1, NP // 4), lambda i: (0, i)),
        out_shape=jax.ShapeDtypeStruct((1, NP), f32),
    )(den2.reshape(NC, NP)).reshape(NP)

    ex2 = lax.bitcast_convert_type(ex_all, i32).reshape(NCH, CB)
    meta = jnp.stack([src2, dst2, typ2, ex2], axis=1).reshape(4 * NCH, CB)
    meta = jnp.pad(meta, ((0, 16), (0, 0)))
    pass2 = functools.partial(
        pl.kernel,
        out_type=jax.ShapeDtypeStruct((NC, NP, DH), f32),
        mesh=mesh,
        scratch_types=(
            [pltpu.VMEM((4, CB), i32)] * 4
            + [pltpu.VMEM((1, CB), i32)] * 4
            + [
                pltpu.VMEM((NP,), f32),
                pltpu.VMEM((R * DH // 2,), i32),
            ]
            + [pltpu.VMEM((CB, DH // 2), i32)] * 4
            + [pltpu.VMEM((CB, DH), f32)] * 4
            + [pltpu.SemaphoreType.DMA] * 12
            + [pltpu.VMEM_SHARED((NP, DH), f32)]
        ),
        compiler_params=pltpu.CompilerParams(needs_layout_passes=False, use_tc_tiling_on_sc=False),
    )(_pass2_body)
    hpart = pass2(meta, winv, ra2f, xa2)

    return jnp.concatenate([hpart[0, :N], hpart[1, :N]], axis=1)
